# baseline (speedup 1.0000x reference)
import functools

import jax
import jax.numpy as jnp
from jax import lax
from jax.experimental import pallas as pl
from jax.experimental.pallas import tpu as pltpu
from jax.experimental.pallas import tpu_sc as plsc

V_TOTAL = 1024
N_FIXED = 256
D = 4096
B = 16 * 1024
_MERGE_BLK = 128
KA = 16
KB = 8


def _merge_body(e_ref, t_ref, o_ref):
    i = pl.program_id(0)
    nfix = N_FIXED // _MERGE_BLK

    @pl.when(i < nfix)
    def _():
        o_ref[...] = e_ref[...]

    @pl.when(i >= nfix)
    def _():
        o_ref[...] = t_ref[...]


def _build_merged(embeddings_weight, trainable_weight):
    nfix = N_FIXED // _MERGE_BLK
    return pl.pallas_call(
        _merge_body,
        grid=(V_TOTAL // _MERGE_BLK,),
        in_specs=[
            pl.BlockSpec((_MERGE_BLK, D), lambda i: (jnp.minimum(i, nfix - 1), 0)),
            pl.BlockSpec((_MERGE_BLK, D), lambda i: (jnp.maximum(i - nfix, 0), 0)),
        ],
        out_specs=pl.BlockSpec((_MERGE_BLK, D), lambda i: (i, 0)),
        out_shape=jax.ShapeDtypeStruct((V_TOTAL, D), jnp.float32),
    )(embeddings_weight, trainable_weight)


def _make_gather(nw, nc, bpw, na, nb):
    mesh = plsc.VectorSubcoreMesh(core_axis_name="c", subcore_axis_name="s")

    @functools.partial(
        pl.kernel,
        mesh=mesh,
        out_type=jax.ShapeDtypeStruct((B, D), jnp.float32),
        scratch_types=[
            pltpu.VMEM((na, KA), jnp.int32),
            pltpu.VMEM((nb, KB), jnp.int32),
            pltpu.VMEM((KA, D), jnp.float32),
            pltpu.VMEM((KB, D), jnp.float32),
        ]
        + [pltpu.SemaphoreType.DMA] * 4,
    )
    def gather(table_hbm, idxa_hbm, idxb_hbm, out_hbm,
               idxa_v, idxb_v, bufa_v, bufb_v, ga_sem, gb_sem, wa_sem, wb_sem):
        wid = lax.axis_index("s") * nc + lax.axis_index("c")
        base = wid * bpw
        pltpu.sync_copy(idxa_hbm.at[wid], idxa_v)
        pltpu.sync_copy(idxb_hbm.at[wid], idxb_v)

        def ga(g):
            return pltpu.make_async_copy(
                table_hbm.at[idxa_v.at[g]], bufa_v, ga_sem)

        def gb(g):
            return pltpu.make_async_copy(
                table_hbm.at[idxb_v.at[g]], bufb_v, gb_sem)

        def wa(g):
            return pltpu.make_async_copy(
                bufa_v, out_hbm.at[pl.ds(base + g * (KA + KB), KA)], wa_sem)

        def wb(g):
            off = base + g * (KA + KB) + jnp.where(g < na, KA, 0)
            return pltpu.make_async_copy(
                bufb_v, out_hbm.at[pl.ds(off, KB)], wb_sem)

        ga(0).start()
        gb(0).start()

        def group(g, carry):
            @pl.when(g < na)
            def _():
                ga(g).wait()
                wa(g).start()

                @pl.when(g + 1 < na)
                def _():
                    wa(g).wait()
                    ga(g + 1).start()

            gb(g).wait()
            wb(g).start()

            @pl.when(g + 1 < nb)
            def _():
                wb(g).wait()
                gb(g + 1).start()
            return carry

        lax.fori_loop(0, nb, group, 0)
        wa(na - 1).wait()
        wb(nb - 1).wait()

    return gather


def kernel(indices, embeddings_weight, trainable_weight):
    info = plsc.get_sparse_core_info()
    nc, ns = info.num_cores, info.num_subcores
    nw = nc * ns
    bpw = B // nw
    na = bpw // (KA + KB)
    nb = na + 1
    assert na * KA + nb * KB == bpw

    merged = _build_merged(embeddings_weight, trainable_weight)
    idx = indices.astype(jnp.int32).reshape(nw, bpw)
    pair = idx[:, : na * (KA + KB)].reshape(nw, na, KA + KB)
    idxa = pair[:, :, :KA]
    idxb_main = pair[:, :, KA:]
    idxb_tail = idx[:, na * (KA + KB):].reshape(nw, 1, KB)
    idxb = jnp.concatenate([idxb_main, idxb_tail], axis=1)

    out = _make_gather(nw, nc, bpw, na, nb)(merged, idxa, idxb)
    return out.reshape(indices.shape[0], indices.shape[1], D)
